# Initial kernel scaffold; baseline (speedup 1.0000x reference)
import functools

import jax
import jax.numpy as jnp
from jax import lax
from jax.experimental import pallas as pl
from jax.experimental.pallas import tpu as pltpu
from jax.experimental.pallas import tpu_sc as plsc

EPS = 1e-06

NC = 2
NS = 16
L = 16
NW = NC * NS


@functools.lru_cache(maxsize=None)
def _build(B: int, T: int):
    assert B % (NW * L) == 0
    bpw = B // NW
    ch = 4096
    nchunk = bpw // ch
    grp = ch // L

    mesh = plsc.VectorSubcoreMesh(core_axis_name="c", subcore_axis_name="s")

    @functools.partial(
        pl.kernel,
        out_type=(
            jax.ShapeDtypeStruct((2 * B,), jnp.float32),
            jax.ShapeDtypeStruct((B,), jnp.float32),
        ),
        mesh=mesh,
        scratch_types=[
            pltpu.VMEM((T * 9,), jnp.float32),
            pltpu.VMEM((ch,), jnp.int32),
            pltpu.VMEM((2 * ch,), jnp.float32),
            pltpu.VMEM((2 * ch,), jnp.float32),
            pltpu.VMEM((ch,), jnp.float32),
        ],
    )
    def homog(h_hbm, xy_hbm, t_hbm, oxy_hbm, ow_hbm, h_v, t_v, xy_v, oxy_v, ow_v):
        wid = lax.axis_index("s") * NC + lax.axis_index("c")
        base = wid * bpw
        pltpu.sync_copy(h_hbm, h_v)

        def chunk_body(ci, carry):
            cbase = base + ci * ch
            pltpu.sync_copy(t_hbm.at[pl.ds(cbase, ch)], t_v)
            pltpu.sync_copy(xy_hbm.at[pl.ds(2 * cbase, 2 * ch)], xy_v)

            def grp_body(g, c2):
                lane = lax.iota(jnp.int32, L)
                i2 = g * (2 * L) + 2 * lane
                x = plsc.load_gather(xy_v, [i2])
                y = plsc.load_gather(xy_v, [i2 + 1])
                tvec = t_v[pl.ds(g * L, L)]
                hb = tvec * 9
                h00 = plsc.load_gather(h_v, [hb])
                h01 = plsc.load_gather(h_v, [hb + 1])
                h02 = plsc.load_gather(h_v, [hb + 2])
                h10 = plsc.load_gather(h_v, [hb + 3])
                h11 = plsc.load_gather(h_v, [hb + 4])
                h12 = plsc.load_gather(h_v, [hb + 5])
                h20 = plsc.load_gather(h_v, [hb + 6])
                h21 = plsc.load_gather(h_v, [hb + 7])
                h22 = plsc.load_gather(h_v, [hb + 8])
                o0 = x * h00 + y * h01 + h02
                o1 = x * h10 + y * h11 + h12
                w = x * h20 + y * h21 + h22
                den = jnp.sign(w) * jnp.maximum(jnp.abs(w), EPS)
                plsc.store_scatter(oxy_v, [i2], o0 / den)
                plsc.store_scatter(oxy_v, [i2 + 1], o1 / den)
                ow_v[pl.ds(g * L, L)] = w
                return c2

            lax.fori_loop(0, grp, grp_body, 0)
            pltpu.sync_copy(oxy_v, oxy_hbm.at[pl.ds(2 * cbase, 2 * ch)])
            pltpu.sync_copy(ow_v, ow_hbm.at[pl.ds(cbase, ch)])
            return carry

        lax.fori_loop(0, nchunk, chunk_body, 0)

    return homog


def kernel(xy, t, H):
    B = xy.shape[0]
    T = H.shape[0]
    homog = _build(B, T)
    oxy, ow = homog(H.reshape(-1), xy.reshape(-1), t)
    return oxy.reshape(B, 2), ow.reshape(B, 1)


# speedup vs baseline: 6.0159x; 6.0159x over previous
import functools

import jax
import jax.numpy as jnp
from jax import lax
from jax.experimental import pallas as pl
from jax.experimental.pallas import tpu as pltpu
from jax.experimental.pallas import tpu_sc as plsc

EPS = 1e-06

NC = 2
NS = 16
L = 16
NW = NC * NS


@functools.lru_cache(maxsize=None)
def _build(B: int, T: int):
    assert B % (NW * L) == 0
    bpw = B // NW
    ch = 4096
    nchunk = bpw // ch
    grp = ch // L

    mesh = plsc.VectorSubcoreMesh(
        core_axis_name="c", subcore_axis_name="s", num_cores=NC, num_subcores=NS
    )

    @functools.partial(
        pl.kernel,
        out_type=(
            jax.ShapeDtypeStruct((2 * B,), jnp.float32),
            jax.ShapeDtypeStruct((B,), jnp.float32),
        ),
        mesh=mesh,
        compiler_params=pltpu.CompilerParams(needs_layout_passes=False),
        scratch_types=[
            pltpu.VMEM((T * 9,), jnp.float32),
            pltpu.VMEM((ch,), jnp.int32),
            pltpu.VMEM((2 * ch,), jnp.float32),
            pltpu.VMEM((2 * ch,), jnp.float32),
            pltpu.VMEM((ch,), jnp.float32),
        ],
    )
    def homog(h_hbm, xy_hbm, t_hbm, oxy_hbm, ow_hbm, h_v, t_v, xy_v, oxy_v, ow_v):
        wid = lax.axis_index("s") * NC + lax.axis_index("c")
        base = wid * bpw
        pltpu.sync_copy(h_hbm, h_v)

        def chunk_body(ci, carry):
            cbase = base + ci * ch
            pltpu.sync_copy(t_hbm.at[pl.ds(cbase, ch)], t_v)
            pltpu.sync_copy(xy_hbm.at[pl.ds(2 * cbase, 2 * ch)], xy_v)

            def grp_body(g, c2):
                lane = lax.iota(jnp.int32, L)
                i2 = g * (2 * L) + 2 * lane
                x = plsc.load_gather(xy_v, [i2])
                y = plsc.load_gather(xy_v, [i2 + 1])
                tvec = t_v[pl.ds(g * L, L)]
                hb = tvec * 9
                h00 = plsc.load_gather(h_v, [hb])
                h01 = plsc.load_gather(h_v, [hb + 1])
                h02 = plsc.load_gather(h_v, [hb + 2])
                h10 = plsc.load_gather(h_v, [hb + 3])
                h11 = plsc.load_gather(h_v, [hb + 4])
                h12 = plsc.load_gather(h_v, [hb + 5])
                h20 = plsc.load_gather(h_v, [hb + 6])
                h21 = plsc.load_gather(h_v, [hb + 7])
                h22 = plsc.load_gather(h_v, [hb + 8])
                o0 = x * h00 + y * h01 + h02
                o1 = x * h10 + y * h11 + h12
                w = x * h20 + y * h21 + h22
                den = jnp.sign(w) * jnp.maximum(jnp.abs(w), EPS)
                plsc.store_scatter(oxy_v, [i2], o0 / den)
                plsc.store_scatter(oxy_v, [i2 + 1], o1 / den)
                ow_v[pl.ds(g * L, L)] = w
                return c2

            lax.fori_loop(0, grp, grp_body, 0)
            pltpu.sync_copy(oxy_v, oxy_hbm.at[pl.ds(2 * cbase, 2 * ch)])
            pltpu.sync_copy(ow_v, ow_hbm.at[pl.ds(cbase, ch)])
            return carry

        lax.fori_loop(0, nchunk, chunk_body, 0)

    return homog


def kernel(xy, t, H):
    B = xy.shape[0]
    T = H.shape[0]
    homog = _build(B, T)
    oxy, ow = homog(H.reshape(-1), xy.reshape(-1), t)
    return oxy.reshape(B, 2), ow.reshape(B, 1)
